# Initial kernel scaffold; baseline (speedup 1.0000x reference)
import jax
import jax.numpy as jnp
from jax.experimental import pallas as pl


def kernel(updates, mask):
    raise NotImplementedError("write your pallas kernel here")


# speedup vs baseline: 2.0618x; 2.0618x over previous
import functools

import jax
import jax.numpy as jnp
from jax import lax
from jax.experimental import pallas as pl
from jax.experimental.pallas import tpu as pltpu
from jax.experimental.pallas import tpu_sc as plsc

B, H, W, C = 8, 112, 112, 96
OUT_H, OUT_W = 2 * H, 2 * W
P = H * W
EB = P * C
N_OUT = B * OUT_H * OUT_W * C

NC, NS = 2, 16
ROWS = OUT_H // 4
SLICE = ROWS * OUT_W * C
TILE_OUT = SLICE // NS
TILE_EL = EB // NS
CHUNK = TILE_EL // 8
VREGS = CHUNK // 16
ZLEN = 4704


def _body(upd_hbm, msk_hbm, out_hbm, shared):
    pl.run_scoped(
        functools.partial(_tile_body, upd_hbm, msk_hbm, out_hbm, shared),
        pltpu.VMEM((CHUNK,), jnp.float32),
        pltpu.VMEM((CHUNK,), jnp.int32),
        pltpu.VMEM((CHUNK,), jnp.int32),
        pltpu.VMEM((ZLEN,), jnp.float32),
    )


def _tile_body(upd_hbm, msk_hbm, out_hbm, shared, upd_v, msk_v, idx_v, zero_v):
    core = lax.axis_index("c")
    sub = lax.axis_index("s")
    iota16 = lax.iota(jnp.int32, 16)

    def zfill(i, carry):
        zero_v[pl.ds(i * 16, 16)] = jnp.zeros((16,), jnp.float32)
        return carry

    lax.fori_loop(0, ZLEN // 16, zfill, 0)

    def one_pass(p, carry):
        b = p // 2
        r = p % 2
        y0 = (2 * r + core) * ROWS

        for j in range(TILE_OUT // ZLEN):
            pltpu.sync_copy(
                zero_v, shared.at[pl.ds(sub * TILE_OUT + j * ZLEN, ZLEN)]
            )
        plsc.subcore_barrier()

        in_base = b * EB + sub * TILE_EL
        for chunk in range(TILE_EL // CHUNK):
            off = in_base + chunk * CHUNK
            pltpu.sync_copy(upd_hbm.at[pl.ds(off, CHUNK)], upd_v)
            pltpu.sync_copy(msk_hbm.at[pl.ds(off, CHUNK)], msk_v)

            def decode(i, carry):
                m = msk_v[pl.ds(i * 16, 16)]
                q = lax.div(m, C)
                y_raw = lax.div(q, OUT_W)
                x = q - y_raw * OUT_W
                y = jnp.minimum(y_raw, OUT_H - 1)
                ok = (y >= y0) & (y < y0 + ROWS)
                cb = (i % (C // 16)) * 16
                li = ((y - y0) * OUT_W + x) * C + cb + iota16
                idx_v[pl.ds(i * 16, 16)] = jnp.where(ok, li, -1)
                return carry

            lax.fori_loop(0, VREGS, decode, 0)
            pltpu.sync_copy(
                upd_v,
                shared.at[plsc.Indices(idx_v, ignored_value=-1)],
                add=True,
            )
        plsc.subcore_barrier()

        out_base = b * (OUT_H * OUT_W * C) + y0 * (OUT_W * C) + sub * TILE_OUT
        pltpu.sync_copy(
            shared.at[pl.ds(sub * TILE_OUT, TILE_OUT)],
            out_hbm.at[pl.ds(out_base, TILE_OUT)],
        )
        return carry

    lax.fori_loop(0, 2 * B, one_pass, 0)


def kernel(updates, mask):
    upd_flat = updates.reshape(-1)
    msk_flat = mask.astype(jnp.int32).reshape(-1)
    run = pl.kernel(
        _body,
        out_type=jax.ShapeDtypeStruct((N_OUT,), jnp.float32),
        mesh=plsc.VectorSubcoreMesh(
            core_axis_name="c", subcore_axis_name="s",
            num_cores=NC, num_subcores=NS,
        ),
        scratch_types=[
            pltpu.VMEM_SHARED((SLICE,), jnp.float32),
        ],
    )
    out_flat = run(upd_flat, msk_flat)
    return out_flat.reshape(B, OUT_H, OUT_W, C)
